# baseline (speedup 1.0000x reference)
import functools

import jax
import jax.numpy as jnp
from jax import lax
from jax.experimental import pallas as pl
from jax.experimental.pallas import tpu as pltpu
from jax.experimental.pallas import tpu_sc as plsc

NC, NS, L = 2, 16, 16
NW = NC * NS
B = 16384
D = 128
BPW = B // NW
C = 128
NCHUNK = BPW // C
DL = D // L

_mesh = plsc.VectorSubcoreMesh(core_axis_name="c", subcore_axis_name="s")


@functools.partial(
    pl.kernel,
    out_type=jax.ShapeDtypeStruct((B,), jnp.float32),
    mesh=_mesh,
    compiler_params=pltpu.CompilerParams(needs_layout_passes=False),
    scratch_types=[
        pltpu.VMEM((NCHUNK, C), jnp.int32),
        pltpu.VMEM((NCHUNK, C), jnp.int32),
        pltpu.VMEM((C, D), jnp.float32),
        pltpu.VMEM((C, D), jnp.float32),
        pltpu.VMEM((BPW,), jnp.float32),
        pltpu.SemaphoreType.DMA,
        pltpu.SemaphoreType.DMA,
    ],
)
def _two_tower_sc(uids_hbm, bids_hbm, utab_hbm, btab_hbm, out_hbm,
                  uid_v, bid_v, urows, brows, out_v, sem_u, sem_b):
    wid = lax.axis_index("s") * NC + lax.axis_index("c")
    base = wid * BPW
    lane = lax.iota(jnp.int32, L)

    for k in range(NCHUNK):
        pltpu.sync_copy(uids_hbm.at[pl.ds(base + k * C, C)], uid_v.at[k])
        pltpu.sync_copy(bids_hbm.at[pl.ds(base + k * C, C)], bid_v.at[k])

    for k in range(NCHUNK):
        cu = pltpu.async_copy(utab_hbm.at[uid_v.at[k]], urows, sem_u)
        cb = pltpu.async_copy(btab_hbm.at[bid_v.at[k]], brows, sem_b)
        cu.wait()
        cb.wait()

        def blk_body(blk, _, k=k):
            out_vec = jnp.zeros((L,), jnp.float32)
            for r in range(L):
                i = blk * L + r
                acc = urows[i, pl.ds(0, L)] * brows[i, pl.ds(0, L)]
                for d in range(1, DL):
                    acc = acc + urows[i, pl.ds(d * L, L)] * brows[i, pl.ds(d * L, L)]
                out_vec = jnp.where(lane == r, jnp.sum(acc), out_vec)
            out_v[pl.ds(k * C + blk * L, L)] = out_vec
            return 0

        lax.fori_loop(0, C // L, blk_body, 0)

    pltpu.sync_copy(out_v, out_hbm.at[pl.ds(base, BPW)])


def kernel(user_ids, banner_ids, user_table, banner_table):
    return _two_tower_sc(user_ids.astype(jnp.int32), banner_ids.astype(jnp.int32),
                         user_table, banner_table)


# speedup vs baseline: 1.1559x; 1.1559x over previous
import functools

import jax
import jax.numpy as jnp
from jax import lax
from jax.experimental import pallas as pl
from jax.experimental.pallas import tpu as pltpu
from jax.experimental.pallas import tpu_sc as plsc

NC, NS, L = 2, 16, 16
NW = NC * NS
B = 16384
D = 128
BPW = B // NW
C = 128
NCHUNK = BPW // C
DL = D // L

_mesh = plsc.VectorSubcoreMesh(core_axis_name="c", subcore_axis_name="s")


@functools.partial(
    pl.kernel,
    out_type=jax.ShapeDtypeStruct((B,), jnp.float32),
    mesh=_mesh,
    compiler_params=pltpu.CompilerParams(needs_layout_passes=False),
    scratch_types=[
        pltpu.VMEM((NCHUNK, C), jnp.int32),
        pltpu.VMEM((NCHUNK, C), jnp.int32),
        pltpu.VMEM((C, D), jnp.float32),
        pltpu.VMEM((C, D), jnp.float32),
        pltpu.VMEM((C, D), jnp.float32),
        pltpu.VMEM((C, D), jnp.float32),
        pltpu.VMEM((BPW,), jnp.float32),
        pltpu.SemaphoreType.DMA,
        pltpu.SemaphoreType.DMA,
        pltpu.SemaphoreType.DMA,
        pltpu.SemaphoreType.DMA,
        pltpu.SemaphoreType.DMA,
    ],
)
def _two_tower_sc(uids_hbm, bids_hbm, utab_hbm, btab_hbm, out_hbm,
                  uid_v, bid_v, u0, b0, u1, b1, out_v,
                  sem_s, sem_u0, sem_b0, sem_u1, sem_b1):
    wid = lax.axis_index("s") * NC + lax.axis_index("c")
    base = wid * BPW
    lane = lax.iota(jnp.int32, L)

    stage = []
    for k in range(NCHUNK):
        stage.append(pltpu.async_copy(uids_hbm.at[pl.ds(base + k * C, C)],
                                      uid_v.at[k], sem_s))
        stage.append(pltpu.async_copy(bids_hbm.at[pl.ds(base + k * C, C)],
                                      bid_v.at[k], sem_s))
    for cp in stage:
        cp.wait()

    ubufs, bbufs = (u0, u1), (b0, b1)
    usems, bsems = (sem_u0, sem_u1), (sem_b0, sem_b1)
    pend = {}

    def start(k):
        pend[k] = (pltpu.async_copy(utab_hbm.at[uid_v.at[k]], ubufs[k % 2], usems[k % 2]),
                   pltpu.async_copy(btab_hbm.at[bid_v.at[k]], bbufs[k % 2], bsems[k % 2]))

    start(0)
    for k in range(NCHUNK):
        if k + 1 < NCHUNK:
            start(k + 1)
        cu, cb = pend.pop(k)
        cu.wait()
        cb.wait()
        urows, brows = ubufs[k % 2], bbufs[k % 2]

        def blk_body(blk, _, k=k, urows=urows, brows=brows):
            out_vec = jnp.zeros((L,), jnp.float32)
            for r in range(L):
                i = blk * L + r
                acc = urows[i, pl.ds(0, L)] * brows[i, pl.ds(0, L)]
                for d in range(1, DL):
                    acc = acc + urows[i, pl.ds(d * L, L)] * brows[i, pl.ds(d * L, L)]
                out_vec = jnp.where(lane == r, jnp.sum(acc), out_vec)
            out_v[pl.ds(k * C + blk * L, L)] = out_vec
            return 0

        lax.fori_loop(0, C // L, blk_body, 0)

    pltpu.sync_copy(out_v, out_hbm.at[pl.ds(base, BPW)])


def kernel(user_ids, banner_ids, user_table, banner_table):
    return _two_tower_sc(user_ids.astype(jnp.int32), banner_ids.astype(jnp.int32),
                         user_table, banner_table)


# speedup vs baseline: 1.3489x; 1.1670x over previous
import functools

import jax
import jax.numpy as jnp
from jax import lax
from jax.experimental import pallas as pl
from jax.experimental.pallas import tpu as pltpu
from jax.experimental.pallas import tpu_sc as plsc

NC, NS, L = 2, 16, 16
NW = NC * NS
B = 16384
D = 128
BPW = B // NW
C = 128
NCHUNK = BPW // C
DL = D // L

_mesh = plsc.VectorSubcoreMesh(core_axis_name="c", subcore_axis_name="s")


@functools.partial(
    pl.kernel,
    out_type=jax.ShapeDtypeStruct((B,), jnp.float32),
    mesh=_mesh,
    compiler_params=pltpu.CompilerParams(needs_layout_passes=False),
    scratch_types=[
        pltpu.VMEM((NCHUNK, C), jnp.int32),
        pltpu.VMEM((NCHUNK, C), jnp.int32),
        pltpu.VMEM((C, D), jnp.float32),
        pltpu.VMEM((C, D), jnp.float32),
        pltpu.VMEM((C, D), jnp.float32),
        pltpu.VMEM((C, D), jnp.float32),
        pltpu.VMEM((BPW,), jnp.float32),
        pltpu.SemaphoreType.DMA,
        pltpu.SemaphoreType.DMA,
        pltpu.SemaphoreType.DMA,
        pltpu.SemaphoreType.DMA,
        pltpu.SemaphoreType.DMA,
    ],
)
def _two_tower_sc(uids_hbm, bids_hbm, utab_hbm, btab_hbm, out_hbm,
                  uid_v, bid_v, u0, b0, u1, b1, out_v,
                  sem_s, sem_u0, sem_b0, sem_u1, sem_b1):
    wid = lax.axis_index("s") * NC + lax.axis_index("c")
    base = wid * BPW
    lane = lax.iota(jnp.int32, L)

    def zero_body(j, _):
        out_v[pl.ds(j * L, L)] = jnp.zeros((L,), jnp.float32)
        return 0

    lax.fori_loop(0, BPW // L, zero_body, 0)

    stage = []
    for k in range(NCHUNK):
        stage.append(pltpu.async_copy(uids_hbm.at[pl.ds(base + k * C, C)],
                                      uid_v.at[k], sem_s))
        stage.append(pltpu.async_copy(bids_hbm.at[pl.ds(base + k * C, C)],
                                      bid_v.at[k], sem_s))
    for cp in stage:
        cp.wait()

    ubufs, bbufs = (u0, u1), (b0, b1)
    usems, bsems = (sem_u0, sem_u1), (sem_b0, sem_b1)
    pend = {}

    def start(k):
        pend[k] = (pltpu.async_copy(utab_hbm.at[uid_v.at[k]], ubufs[k % 2], usems[k % 2]),
                   pltpu.async_copy(btab_hbm.at[bid_v.at[k]], bbufs[k % 2], bsems[k % 2]))

    start(0)
    for k in range(NCHUNK):
        if k + 1 < NCHUNK:
            start(k + 1)
        cu, cb = pend.pop(k)
        cu.wait()
        cb.wait()
        urows, brows = ubufs[k % 2], bbufs[k % 2]

        def blk_body(blk, _, k=k, urows=urows, brows=brows):
            for r in range(L):
                i = blk * L + r
                acc = urows[i, pl.ds(0, L)] * brows[i, pl.ds(0, L)]
                for d in range(1, DL):
                    acc = acc + urows[i, pl.ds(d * L, L)] * brows[i, pl.ds(d * L, L)]
                gi = jnp.full((L,), k * C, jnp.int32) + i
                plsc.addupdate_scatter(out_v, [gi], acc)
            return 0

        lax.fori_loop(0, C // L, blk_body, 0)

    pltpu.sync_copy(out_v, out_hbm.at[pl.ds(base, BPW)])


def kernel(user_ids, banner_ids, user_table, banner_table):
    return _two_tower_sc(user_ids.astype(jnp.int32), banner_ids.astype(jnp.int32),
                         user_table, banner_table)


# speedup vs baseline: 1.4984x; 1.1109x over previous
import functools

import jax
import jax.numpy as jnp
from jax import lax
from jax.experimental import pallas as pl
from jax.experimental.pallas import tpu as pltpu
from jax.experimental.pallas import tpu_sc as plsc

NC, NS, L = 2, 16, 16
NW = NC * NS
B = 16384
D = 128
BPW = B // NW
C = 128
NCHUNK = BPW // C
DL = D // L

_mesh = plsc.VectorSubcoreMesh(core_axis_name="c", subcore_axis_name="s")


@functools.partial(
    pl.kernel,
    out_type=jax.ShapeDtypeStruct((B,), jnp.float32),
    mesh=_mesh,
    compiler_params=pltpu.CompilerParams(needs_layout_passes=False),
    scratch_types=[
        pltpu.VMEM((NCHUNK, C), jnp.int32),
        pltpu.VMEM((NCHUNK, C), jnp.int32),
        pltpu.VMEM((C, D), jnp.float32),
        pltpu.VMEM((C, D), jnp.float32),
        pltpu.VMEM((C, D), jnp.float32),
        pltpu.VMEM((C, D), jnp.float32),
        pltpu.VMEM((L * BPW,), jnp.float32),
        pltpu.VMEM((BPW,), jnp.float32),
        pltpu.SemaphoreType.DMA,
        pltpu.SemaphoreType.DMA,
        pltpu.SemaphoreType.DMA,
        pltpu.SemaphoreType.DMA,
        pltpu.SemaphoreType.DMA,
    ],
)
def _two_tower_sc(uids_hbm, bids_hbm, utab_hbm, btab_hbm, out_hbm,
                  uid_v, bid_v, u0, b0, u1, b1, q_t, out_v,
                  sem_s, sem_u0, sem_b0, sem_u1, sem_b1):
    wid = lax.axis_index("s") * NC + lax.axis_index("c")
    base = wid * BPW
    lane = lax.iota(jnp.int32, L)

    stage = []
    for k in range(NCHUNK):
        stage.append(pltpu.async_copy(uids_hbm.at[pl.ds(base + k * C, C)],
                                      uid_v.at[k], sem_s))
        stage.append(pltpu.async_copy(bids_hbm.at[pl.ds(base + k * C, C)],
                                      bid_v.at[k], sem_s))
    for cp in stage:
        cp.wait()

    ubufs, bbufs = (u0, u1), (b0, b1)
    usems, bsems = (sem_u0, sem_u1), (sem_b0, sem_b1)
    pend = {}

    def start(k):
        pend[k] = (pltpu.async_copy(utab_hbm.at[uid_v.at[k]], ubufs[k % 2], usems[k % 2]),
                   pltpu.async_copy(btab_hbm.at[bid_v.at[k]], bbufs[k % 2], bsems[k % 2]))

    start(0)
    for k in range(NCHUNK):
        if k + 1 < NCHUNK:
            start(k + 1)
        cu, cb = pend.pop(k)
        cu.wait()
        cb.wait()
        urows, brows = ubufs[k % 2], bbufs[k % 2]

        def blk_body(blk, _, k=k, urows=urows, brows=brows):
            for r in range(L):
                i = blk * L + r
                acc = urows[i, pl.ds(0, L)] * brows[i, pl.ds(0, L)]
                for d in range(1, DL):
                    acc = acc + urows[i, pl.ds(d * L, L)] * brows[i, pl.ds(d * L, L)]
                gi = lane * BPW + (k * C + i)
                plsc.store_scatter(q_t, [gi], acc)
            return 0

        lax.fori_loop(0, C // L, blk_body, 0)

    def red_body(j, _):
        s = j * L
        acc = q_t[pl.ds(s, L)]
        for l in range(1, L):
            acc = acc + q_t[pl.ds(l * BPW + s, L)]
        out_v[pl.ds(s, L)] = acc
        return 0

    lax.fori_loop(0, BPW // L, red_body, 0)

    pltpu.sync_copy(out_v, out_hbm.at[pl.ds(base, BPW)])


def kernel(user_ids, banner_ids, user_table, banner_table):
    return _two_tower_sc(user_ids.astype(jnp.int32), banner_ids.astype(jnp.int32),
                         user_table, banner_table)
